# baseline (speedup 1.0000x reference)
import math

import jax
import jax.numpy as jnp
from jax import lax
from jax.experimental import pallas as pl
from jax.experimental.pallas import tpu as pltpu
from jax.experimental.pallas import tpu_sc as plsc

_BASE = 2
_REP = 8
_HEAD = 2


def _sc_body(emb_hbm, out_hbm, pat, idx1, idx2, g1sem, g2sem, sem):
    cid = lax.axis_index("c")
    sid = lax.axis_index("s")
    wid = sid * 2 + cid
    max_level, t_total, _ = out_hbm.shape
    chunk = t_total // 32
    t0 = wid * chunk
    nstream = chunk // _REP
    head_rows = _HEAD * _REP

    def level_vals(j):
        lvl = j >> 3
        par = j & 1
        v = (lvl << 1) + par * (1 - (lvl & 1))
        return jnp.where(lvl < max_level, v, 0)

    for c0 in range(0, head_rows, 16):
        idx1[pl.ds(c0, 16)] = level_vals(c0 + lax.iota(jnp.int32, 16))
    h1 = pltpu.async_copy(emb_hbm.at[idx1], pat.at[pl.ds(0, head_rows)],
                          g1sem)
    for c0 in range(0, idx2.shape[0], 16):
        idx2[pl.ds(c0, 16)] = level_vals(
            head_rows + c0 + lax.iota(jnp.int32, 16))
    h2 = pltpu.async_copy(
        emb_hbm.at[idx2],
        pat.at[pl.ds(head_rows, idx2.shape[0])], g2sem)

    pending = []

    def emit_level(l):
        src = pat.at[pl.ds(l * _REP, _REP)]
        for k in range(nstream):
            pending.append(pltpu.async_copy(
                src, out_hbm.at[l, pl.ds(t0 + k * _REP, _REP)], sem))

    h1.wait()
    for l in range(_HEAD):
        emit_level(l)
    h2.wait()
    for l in range(_HEAD, max_level):
        emit_level(l)
    for h in pending:
        h.wait()


def kernel(x, emb):
    B, T = x.shape
    del B
    max_level = int(math.ceil(math.log(T, _BASE)))
    d = emb.shape[1]
    tail_rows = -(-(max_level - _HEAD) * _REP // 16) * 16
    n_rows = _HEAD * _REP + tail_rows

    mesh = plsc.VectorSubcoreMesh(core_axis_name="c", subcore_axis_name="s")
    k = pl.kernel(
        _sc_body,
        out_type=jax.ShapeDtypeStruct((max_level, T, d), emb.dtype),
        mesh=mesh,
        scratch_types=[
            pltpu.VMEM((n_rows, d), emb.dtype),
            pltpu.VMEM((_HEAD * _REP,), jnp.int32),
            pltpu.VMEM((tail_rows,), jnp.int32),
            pltpu.SemaphoreType.DMA,
            pltpu.SemaphoreType.DMA,
            pltpu.SemaphoreType.DMA,
        ],
    )
    return k(emb)


# speedup vs baseline: 1.0948x; 1.0948x over previous
import math

import jax
import jax.numpy as jnp
from jax import lax
from jax.experimental import pallas as pl
from jax.experimental.pallas import tpu as pltpu
from jax.experimental.pallas import tpu_sc as plsc

_BASE = 2
_REP = 8


def _sc_body(emb_hbm, out_hbm, pat, idx, gsem, sem):
    cid = lax.axis_index("c")
    sid = lax.axis_index("s")
    wid = sid * 2 + cid
    max_level, t_total, _ = out_hbm.shape
    chunk = t_total // 32
    t0 = wid * chunk
    nstream = chunk // _REP
    npad = idx.shape[0]

    for c0 in range(0, npad, 16):
        j = c0 + lax.iota(jnp.int32, 16)
        lvl = j >> 3
        par = j & 1
        vals = (lvl << 1) + par * (1 - (lvl & 1))
        vals = jnp.where(lvl < max_level, vals, 0)
        idx[pl.ds(c0, 16)] = vals
    pltpu.async_copy(emb_hbm.at[idx], pat, gsem).wait()

    pending = []
    for l in range(max_level):
        src = pat.at[pl.ds(l * _REP, _REP)]
        for k in range(nstream):
            h = pltpu.async_copy(
                src, out_hbm.at[l, pl.ds(t0 + k * _REP, _REP)], sem)
            pending.append(h)
    for h in pending:
        h.wait()


def kernel(x, emb):
    B, T = x.shape
    del B
    max_level = int(math.ceil(math.log(T, _BASE)))
    d = emb.shape[1]
    npad = -(-max_level * _REP // 16) * 16

    mesh = plsc.VectorSubcoreMesh(core_axis_name="c", subcore_axis_name="s")
    k = pl.kernel(
        _sc_body,
        out_type=jax.ShapeDtypeStruct((max_level, T, d), emb.dtype),
        mesh=mesh,
        scratch_types=[
            pltpu.VMEM((npad, d), emb.dtype),
            pltpu.VMEM((npad,), jnp.int32),
            pltpu.SemaphoreType.DMA,
            pltpu.SemaphoreType.DMA,
        ],
    )
    return k(emb)
